# Initial kernel scaffold; baseline (speedup 1.0000x reference)
import jax
import jax.numpy as jnp
from jax.experimental import pallas as pl


def kernel(spherical_displacement, grid_vertices, D, H, W):
    raise NotImplementedError("write your pallas kernel here")


# speedup vs baseline: 2.1074x; 2.1074x over previous
import jax
import jax.numpy as jnp
from jax import lax
from jax.experimental import pallas as pl

_K = 4096
_N = 64 * 64 * 64
_BQ = 1024
_KT = 512
_G = 64


def _nn_body(qd_ref, kd_ref, tab_ref, out_ref):
    t = qd_ref[0:1, :]
    p = qd_ref[1:2, :]
    rho = qd_ref[2:3, :]
    B = t.shape[1]

    best_m = jnp.full((1, B), jnp.inf, jnp.float32)
    best_i = jnp.zeros((1, B), jnp.int32)
    for kt in range(_K // _KT):
        tg = kd_ref[kt * _KT:(kt + 1) * _KT, 0:1]
        pg = kd_ref[kt * _KT:(kt + 1) * _KT, 1:2]
        dt = t - tg
        dp = p - pg
        dist = dt * dt + dp * dp
        m = jnp.min(dist, axis=0, keepdims=True)
        iota = lax.broadcasted_iota(jnp.int32, (_KT, B), 0) + kt * _KT
        idx = jnp.min(jnp.where(dist == m, iota, _K), axis=0, keepdims=True)
        upd = m < best_m
        best_m = jnp.where(upd, m, best_m)
        best_i = jnp.where(upd, idx, best_i)

    hi = best_i >> 6
    lo = best_i & 63
    oh = (lax.broadcasted_iota(jnp.int32, (_G, B), 0) == hi).astype(jnp.float32)
    sel = lax.dot_general(tab_ref[...], oh, (((1,), (0,)), ((), ())),
                          preferred_element_type=jnp.float32)
    mask = ((lax.broadcasted_iota(jnp.int32, (256, B), 0) >> 2) == lo)
    dsel = jnp.sum((sel * mask.astype(jnp.float32)).reshape(_G, 4, B), axis=0)
    d_rho = dsel[0:1]
    d_th = dsel[1:2]
    d_ph = dsel[2:3]

    st = jnp.sin(t)
    ct = jnp.cos(t)
    sp = jnp.sin(p)
    cp = jnp.cos(p)
    a = rho * d_th
    b = rho * st * d_ph
    ox = d_rho * (st * cp) + a * (ct * cp) - b * sp
    oy = d_rho * (st * sp) + a * (ct * sp) + b * cp
    oz = d_rho * ct - a * st
    out_ref[...] = jnp.concatenate([ox, oy, oz], axis=0)


def kernel(spherical_displacement, grid_vertices, D, H, W):
    gx, gy, gz = grid_vertices.T
    r_ = jnp.sqrt(gx ** 2 + gy ** 2 + gz ** 2)
    theta_g = jnp.arccos(gz / jnp.maximum(r_, 1e-6))
    phi_g = jnp.arctan2(gy, gx)

    coords = jnp.stack(
        jnp.meshgrid(jnp.arange(64), jnp.arange(64), jnp.arange(64), indexing="ij"),
        axis=-1,
    ).astype(jnp.float32)
    center = jnp.stack(
        [(D - 1) / 2.0, (H - 1) / 2.0, (W - 1) / 2.0]
    ).astype(jnp.float32)
    cc = coords.reshape(-1, 3) - center
    x, y, z = cc.T
    rho = jnp.linalg.norm(cc, axis=1)
    theta = jnp.arccos(z / jnp.maximum(rho, 1e-6))
    phi = jnp.arctan2(y, x)

    qd = jnp.stack([theta, phi, rho, jnp.zeros_like(rho)], axis=0)
    kd = jnp.stack([theta_g, phi_g], axis=1)
    tab = jnp.pad(spherical_displacement, ((0, 0), (0, 1)))
    tab = tab.reshape(_G, 64, 4).transpose(1, 2, 0).reshape(256, _G)

    out = pl.pallas_call(
        _nn_body,
        grid=(_N // _BQ,),
        in_specs=[
            pl.BlockSpec((4, _BQ), lambda i: (0, i)),
            pl.BlockSpec((_K, 2), lambda i: (0, 0)),
            pl.BlockSpec((256, _G), lambda i: (0, 0)),
        ],
        out_specs=pl.BlockSpec((3, _BQ), lambda i: (0, i)),
        out_shape=jax.ShapeDtypeStruct((3, _N), jnp.float32),
    )(qd, kd, tab)
    return out.reshape(3, 64, 64, 64)
